# baseline (speedup 1.0000x reference)
import functools

import jax
import jax.numpy as jnp
from jax import lax
from jax.experimental import pallas as pl
from jax.experimental.pallas import tpu as pltpu
from jax.experimental.pallas import tpu_sc as plsc

VOCAB = 1000000
EMBED = 64
BATCH = 16384
SEQ = 50

NC = 2
NS = 16
NW = NC * NS

BPW = BATCH // NW
BLK = 128
BLKS_PER_S = BPW // BLK
NBLK = SEQ * BLKS_PER_S
L = 16


@functools.partial(
    pl.kernel,
    out_type=jax.ShapeDtypeStruct((SEQ, EMBED, BATCH), jnp.float32),
    mesh=plsc.VectorSubcoreMesh(core_axis_name="c", subcore_axis_name="s"),
    scratch_types=[
        pltpu.VMEM((SEQ, BPW), jnp.int32),
        pltpu.VMEM((2, BLK, 128), jnp.float32),
        pltpu.VMEM((2, EMBED, BLK), jnp.float32),
        pltpu.VMEM((2, BLK), jnp.int32),
        pltpu.VMEM((2, BLK), jnp.int32),
        pltpu.SemaphoreType.DMA((2,)),
        pltpu.SemaphoreType.DMA((2,)),
    ],
    compiler_params=pltpu.CompilerParams(needs_layout_passes=False),
)
def _embed_lookup(tab2, idx_t, out, idxall, rows, stage, idxhalf, colb,
                  gsem, ssem):
    wid = lax.axis_index("s") * NC + lax.axis_index("c")
    w0 = wid * BPW
    iota16 = lax.iota(jnp.int32, L)
    rowvecs = [iota16 + g * L for g in range(BLK // L)]

    def stage_idx(s, carry):
        pltpu.sync_copy(idx_t.at[s, pl.ds(w0, BPW)], idxall.at[s])
        return carry

    lax.fori_loop(0, SEQ, stage_idx, 0)

    def prep(k, p):
        s = k // BLKS_PER_S
        c0 = (k % BLKS_PER_S) * BLK
        for g in range(BLK // L):
            v = idxall[s, pl.ds(c0 + g * L, L)]
            idxhalf[p, pl.ds(g * L, L)] = v >> 1
            colb[p, pl.ds(g * L, L)] = (v & 1) * EMBED
        pltpu.async_copy(tab2.at[idxhalf.at[p]], rows.at[p], gsem.at[p])

    def wait_gather(p):
        pltpu.make_async_copy(
            tab2.at[idxhalf.at[p]], rows.at[p], gsem.at[p]).wait()

    def fire_store(k, p):
        s = k // BLKS_PER_S
        b0 = w0 + (k % BLKS_PER_S) * BLK
        pltpu.async_copy(
            stage.at[p], out.at[s, :, pl.ds(b0, BLK)], ssem.at[p])

    def wait_store(p):
        pltpu.make_async_copy(
            stage.at[p], out.at[0, :, pl.ds(0, BLK)], ssem.at[p]).wait()

    EU = 8

    def transpose(p):
        cb = [colb[p, pl.ds(g * L, L)] for g in range(BLK // L)]

        def ebody(eo, carry):
            for q in range(EU):
                e = eo * EU + q
                for g in range(BLK // L):
                    vals = plsc.load_gather(
                        rows.at[p], [rowvecs[g], cb[g] + e])
                    stage[p, e, pl.ds(g * L, L)] = vals
            return carry

        lax.fori_loop(0, EMBED // EU, ebody, 0)

    for p in range(2):
        prep(p, p)

    def body(kk, carry):
        for p in range(2):
            k = 2 * kk + p
            pl.when(kk > 0)(lambda p=p: wait_store(p))
            wait_gather(p)
            transpose(p)
            fire_store(k, p)
            pl.when(kk < NBLK // 2 - 1)(lambda k=k, p=p: prep(k + 2, p))
        return carry

    lax.fori_loop(0, NBLK // 2, body, 0)
    for p in range(2):
        wait_store(p)


def kernel(src_indices, tgt_indices, src_table, tgt_table):
    src_idx = src_indices.T.astype(jnp.int32)
    tgt_idx = tgt_indices.T.astype(jnp.int32)
    src_tab = src_table.reshape(VOCAB // 2, 2 * EMBED)
    tgt_tab = tgt_table.reshape(VOCAB // 2, 2 * EMBED)
    src_out = _embed_lookup(src_tab, src_idx)
    tgt_out = _embed_lookup(tgt_tab, tgt_idx)
    return (
        jnp.transpose(src_out, (2, 0, 1)),
        jnp.transpose(tgt_out, (2, 0, 1)),
    )


# speedup vs baseline: 1.1717x; 1.1717x over previous
import functools

import jax
import jax.numpy as jnp
from jax import lax
from jax.experimental import pallas as pl
from jax.experimental.pallas import tpu as pltpu
from jax.experimental.pallas import tpu_sc as plsc

VOCAB = 1000000
EMBED = 64
BATCH = 16384
SEQ = 50

NC = 2
NS = 16
NW = NC * NS

BPW = BATCH // NW
BLK = 128
BLKS_PER_S = BPW // BLK
NBLK = SEQ * BLKS_PER_S
L = 16


@functools.partial(
    pl.kernel,
    out_type=jax.ShapeDtypeStruct((SEQ, EMBED, BATCH), jnp.float32),
    mesh=plsc.VectorSubcoreMesh(core_axis_name="c", subcore_axis_name="s"),
    scratch_types=[
        pltpu.VMEM((SEQ, BPW), jnp.int32),
        pltpu.VMEM((2, BLK, 128), jnp.float32),
        pltpu.VMEM((2, EMBED, BLK), jnp.float32),
        pltpu.VMEM((2, BLK), jnp.int32),
        pltpu.VMEM((2, BLK), jnp.int32),
        pltpu.SemaphoreType.DMA((2,)),
        pltpu.SemaphoreType.DMA((2,)),
    ],
    compiler_params=pltpu.CompilerParams(needs_layout_passes=False),
)
def _embed_lookup(tab2, idx_t, out, idxall, rows, stage, idxhalf, colb,
                  gsem, ssem):
    wid = lax.axis_index("s") * NC + lax.axis_index("c")
    w0 = wid * BPW
    iota16 = lax.iota(jnp.int32, L)
    rowvecs = [iota16 + g * L for g in range(BLK // L)]

    def stage_idx(s, carry):
        pltpu.sync_copy(idx_t.at[s, pl.ds(w0, BPW)], idxall.at[s])
        return carry

    lax.fori_loop(0, SEQ, stage_idx, 0)

    def prep(k, p):
        s = k // BLKS_PER_S
        c0 = (k % BLKS_PER_S) * BLK
        for g in range(BLK // L):
            v = idxall[s, pl.ds(c0 + g * L, L)]
            idxhalf[p, pl.ds(g * L, L)] = v >> 1
            colb[p, pl.ds(g * L, L)] = (v & 1) * EMBED
        pltpu.async_copy(tab2.at[idxhalf.at[p]], rows.at[p], gsem.at[p])

    def wait_gather(p):
        pltpu.make_async_copy(
            tab2.at[idxhalf.at[p]], rows.at[p], gsem.at[p]).wait()

    def fire_store(k, p):
        s = k // BLKS_PER_S
        b0 = w0 + (k % BLKS_PER_S) * BLK
        pltpu.async_copy(
            stage.at[p], out.at[s, :, pl.ds(b0, BLK)], ssem.at[p])

    def wait_store(p):
        pltpu.make_async_copy(
            stage.at[p], out.at[0, :, pl.ds(0, BLK)], ssem.at[p]).wait()

    EU = 8

    def transpose(p):
        cb = [colb[p, pl.ds(g * L, L)] for g in range(BLK // L)]

        def ebody(eo, carry):
            for q in range(EU):
                e = eo * EU + q
                vals = [
                    plsc.load_gather(rows.at[p], [rowvecs[g], cb[g] + e])
                    for g in range(BLK // L)
                ]
                for g in range(BLK // L):
                    stage[p, e, pl.ds(g * L, L)] = vals[g]
            return carry

        lax.fori_loop(0, EMBED // EU, ebody, 0)

    for p in range(2):
        prep(p, p)

    def body(kk, carry):
        for p in range(2):
            k = 2 * kk + p
            pl.when(kk > 0)(lambda p=p: wait_store(p))
            wait_gather(p)
            transpose(p)
            fire_store(k, p)
            pl.when(kk < NBLK // 2 - 1)(lambda k=k, p=p: prep(k + 2, p))
        return carry

    lax.fori_loop(0, NBLK // 2, body, 0)
    for p in range(2):
        wait_store(p)


def kernel(src_indices, tgt_indices, src_table, tgt_table):
    src_idx = src_indices.T.astype(jnp.int32)
    tgt_idx = tgt_indices.T.astype(jnp.int32)
    src_tab = src_table.reshape(VOCAB // 2, 2 * EMBED)
    tgt_tab = tgt_table.reshape(VOCAB // 2, 2 * EMBED)
    src_out = _embed_lookup(src_tab, src_idx)
    tgt_out = _embed_lookup(tgt_tab, tgt_idx)
    return (
        jnp.transpose(src_out, (2, 0, 1)),
        jnp.transpose(tgt_out, (2, 0, 1)),
    )
